# baseline (speedup 1.0000x reference)
import functools

import jax
import jax.numpy as jnp
from jax import lax
from jax.experimental import pallas as pl
from jax.experimental.pallas import tpu as pltpu
from jax.experimental.pallas import tpu_sc as plsc

N = 10000
E = 320000
D_IN = 128
D_HID = 16
N_CLS = 40

NC = 2
NS = 16
NW = NC * NS
EPT = E // NW
RPT = N // NS

SUBR = 80
CHE = 2000
NSUBC = CHE // SUBR
NCHE = EPT // CHE
NROW = E // SUBR
DROW = EPT // SUBR

_mesh = plsc.VectorSubcoreMesh(core_axis_name="c", subcore_axis_name="s")
_sc_params = pltpu.CompilerParams(
    needs_layout_passes=False, use_tc_tiling_on_sc=False)


@functools.partial(
    pl.kernel,
    out_type=jax.ShapeDtypeStruct((NW, N), jnp.float32),
    mesh=_mesh,
    compiler_params=_sc_params,
    scratch_types=[
        pltpu.VMEM((DROW, SUBR), jnp.int32),
        pltpu.VMEM((DROW, SUBR), jnp.float32),
        pltpu.VMEM((N,), jnp.float32),
    ],
)
def _deg_kernel(dst_hbm, ew_hbm, out_hbm, dst_v, ew_v, deg_v):
    c = lax.axis_index("c")
    s = lax.axis_index("s")
    wid = c * NS + s
    pltpu.sync_copy(dst_hbm.at[pl.ds(wid * DROW, DROW)], dst_v)
    pltpu.sync_copy(ew_hbm.at[pl.ds(wid * DROW, DROW)], ew_v)

    def zero_body(i, carry):
        deg_v[pl.ds(i * 16, 16)] = jnp.zeros((16,), jnp.float32)
        return carry

    lax.fori_loop(0, N // 16, zero_body, 0, unroll=4)

    def body(r, carry):
        for q in range(SUBR // 16):
            idx = dst_v[r, pl.ds(q * 16, 16)]
            w = ew_v[r, pl.ds(q * 16, 16)]
            plsc.addupdate_scatter(deg_v, [idx], w)
        return carry

    lax.fori_loop(0, DROW, body, 0, unroll=2)
    pltpu.sync_copy(deg_v, out_hbm.at[wid])


@functools.partial(
    pl.kernel,
    out_type=jax.ShapeDtypeStruct((NC, N, D_HID), jnp.float32),
    mesh=_mesh,
    compiler_params=_sc_params,
    scratch_types=[
        pltpu.VMEM((2, NSUBC, SUBR), jnp.int32),
        pltpu.VMEM((2, NSUBC, SUBR), jnp.int32),
        pltpu.VMEM((2, NSUBC, SUBR), jnp.float32),
        pltpu.VMEM((2, CHE, D_HID), jnp.float32),
        pltpu.VMEM((RPT, D_HID), jnp.float32),
        pltpu.VMEM((N,), jnp.float32),
        pltpu.VMEM_SHARED((N, D_HID), jnp.float32),
        pltpu.VMEM_SHARED((N, D_HID), jnp.float32),
        pltpu.SemaphoreType.DMA,
        pltpu.SemaphoreType.DMA,
        pltpu.SemaphoreType.DMA,
    ],
)
def _agg_kernel(hs_hbm, dinv_hbm, src_hbm, dst_hbm, ew_hbm, out_hbm,
                srcb, dstb, ewb, rows, zbuf, dinv_v, acc_s, hs_s,
                seme, semg, sems):
    c = lax.axis_index("c")
    s = lax.axis_index("s")
    wid = c * NS + s

    stage = pltpu.async_copy(hs_hbm.at[pl.ds(s * RPT, RPT)],
                             hs_s.at[pl.ds(s * RPT, RPT)], semg)
    dstage = pltpu.async_copy(dinv_hbm, dinv_v, semg)

    def zero_body(i, carry):
        zbuf[i, :] = jnp.zeros((D_HID,), jnp.float32)
        return carry

    lax.fori_loop(0, RPT, zero_body, 0, unroll=8)
    pltpu.sync_copy(zbuf, acc_s.at[pl.ds(s * RPT, RPT)])
    stage.wait()
    dstage.wait()
    plsc.subcore_barrier()

    row0 = wid * (NCHE * NSUBC)

    def fire_edges(ch, buf):
        rb = row0 + ch * NSUBC
        return [
            pltpu.async_copy(src_hbm.at[pl.ds(rb, NSUBC)], srcb.at[buf], seme),
            pltpu.async_copy(dst_hbm.at[pl.ds(rb, NSUBC)], dstb.at[buf], seme),
            pltpu.async_copy(ew_hbm.at[pl.ds(rb, NSUBC)], ewb.at[buf], seme),
        ]

    edge_pend = {0: fire_edges(0, 0), 1: None}
    scat_pend = {0: [], 1: []}
    for ch in range(NCHE):
        buf = ch % 2
        for d in edge_pend[buf]:
            d.wait()
        for d in scat_pend[buf]:
            d.wait()
        scat_pend[buf] = []
        gathers = [
            pltpu.async_copy(hs_s.at[srcb.at[buf, j]],
                             rows.at[buf, pl.ds(j * SUBR, SUBR)], semg)
            for j in range(NSUBC)
        ]
        if ch + 1 < NCHE:
            for d in scat_pend[1 - buf]:
                d.wait()
            scat_pend[1 - buf] = []
            edge_pend[1 - buf] = fire_edges(ch + 1, 1 - buf)
        for g in gathers:
            g.wait()

        def scale_body(j, carry):
            for q in range(SUBR // 16):
                ew16 = ewb[buf, j, pl.ds(q * 16, 16)]
                sv16 = srcb[buf, j, pl.ds(q * 16, 16)]
                f16 = ew16 * plsc.load_gather(dinv_v, [sv16])
                for e in range(16):
                    r = j * SUBR + q * 16 + e
                    w = jnp.broadcast_to(f16[e], (D_HID,))
                    rows[buf, r, :] = rows[buf, r, :] * w
            return carry

        lax.fori_loop(0, NSUBC, scale_body, 0)

        plsc.subcore_barrier()
        scat_pend[buf] = [
            pltpu.async_copy(rows.at[buf, pl.ds(j * SUBR, SUBR)],
                             acc_s.at[dstb.at[buf, j]], sems, add=True)
            for j in range(NSUBC)
        ]

    for d in scat_pend[0] + scat_pend[1]:
        d.wait()
    plsc.subcore_barrier()
    pltpu.sync_copy(acc_s.at[pl.ds(s * RPT, RPT)], zbuf)
    pltpu.sync_copy(zbuf, out_hbm.at[c, pl.ds(s * RPT, RPT)])


def _mm_body(x_ref, w_ref, h_ref):
    h_ref[...] = jnp.dot(x_ref[...], w_ref[...],
                         preferred_element_type=jnp.float32)


@jax.jit
def _mm(x, W1):
    return pl.pallas_call(
        _mm_body,
        out_shape=jax.ShapeDtypeStruct((N, D_HID), jnp.float32),
    )(x, W1)


def _dinvk_body(degp_ref, dinv_ref, dcol_ref):
    deg = 1.0 + jnp.sum(degp_ref[...], axis=0)
    di = lax.rsqrt(deg)
    dinv_ref[...] = di[None, :]
    dcol_ref[...] = di[:, None]


@jax.jit
def _dinvk(degp):
    return pl.pallas_call(
        _dinvk_body,
        out_shape=(
            jax.ShapeDtypeStruct((1, N), jnp.float32),
            jax.ShapeDtypeStruct((N, 1), jnp.float32),
        ),
    )(degp)


def _post1_body(p_ref, h_ref, dinv_ref, b_ref, out_ref):
    di = dinv_ref[...]
    t = p_ref[0] + p_ref[1] + di * h_ref[...]
    out_ref[...] = jax.nn.relu(di * t + b_ref[...])


@jax.jit
def _post1(parts, hs1, dinv, b1r):
    return pl.pallas_call(
        _post1_body,
        out_shape=jax.ShapeDtypeStruct((N, D_HID), jnp.float32),
    )(parts, hs1, dinv, b1r)


def _final_body(p_ref, h_ref, dinv_ref, w_ref, b_ref, out_ref):
    di = dinv_ref[...]
    agg = di * (p_ref[0] + p_ref[1] + di * h_ref[...])
    logits = jnp.dot(agg, w_ref[...], preferred_element_type=jnp.float32)
    logits = logits + b_ref[...]
    z = logits - jnp.max(logits, axis=1, keepdims=True)
    out_ref[...] = z - jnp.log(jnp.sum(jnp.exp(z), axis=1, keepdims=True))


@jax.jit
def _final(parts, hs2, dinv, W2, b2r):
    return pl.pallas_call(
        _final_body,
        out_shape=jax.ShapeDtypeStruct((N, N_CLS), jnp.float32),
    )(parts, hs2, dinv, W2, b2r)


def kernel(x, edge_index, edge_weight, W1, b1, W2, b2):
    src = edge_index[0]
    dst = edge_index[1]
    srcg = src.reshape(NROW, SUBR)
    dstg = dst.reshape(NROW, SUBR)
    ewg = edge_weight.reshape(NROW, SUBR)
    degp = _deg_kernel(dstg, ewg)
    h1 = _mm(x, W1)
    dinv_row, dinv_col = _dinvk(degp)
    dinv_flat = dinv_row.reshape(N)
    parts1 = _agg_kernel(h1, dinv_flat, srcg, dstg, ewg)
    out1 = _post1(parts1, h1, dinv_col, b1.reshape(1, D_HID))
    parts2 = _agg_kernel(out1, dinv_flat, srcg, dstg, ewg)
    return _final(parts2, out1, dinv_col, W2, b2.reshape(1, N_CLS))


# speedup vs baseline: 1.1838x; 1.1838x over previous
import functools

import jax
import jax.numpy as jnp
from jax import lax
from jax.experimental import pallas as pl
from jax.experimental.pallas import tpu as pltpu
from jax.experimental.pallas import tpu_sc as plsc

N = 10000
E = 320000
D_IN = 128
D_HID = 16
N_CLS = 40

NC = 2
NS = 16
NW = NC * NS
EPT = E // NW
RPT = N // NS

SUBR = 128
CHE = 2560
NSUBC = CHE // SUBR
EPTP = 10240
NCHE = EPTP // CHE
E_PAD = NW * EPTP
NROW = E_PAD // SUBR

_mesh = plsc.VectorSubcoreMesh(core_axis_name="c", subcore_axis_name="s")
_sc_params = pltpu.CompilerParams(
    needs_layout_passes=False, use_tc_tiling_on_sc=False)


@functools.partial(
    pl.kernel,
    out_type=jax.ShapeDtypeStruct((NW, N), jnp.float32),
    mesh=_mesh,
    compiler_params=_sc_params,
    scratch_types=[
        pltpu.VMEM((EPT,), jnp.int32),
        pltpu.VMEM((EPT,), jnp.float32),
        pltpu.VMEM((N,), jnp.float32),
    ],
)
def _deg_kernel(dst_hbm, ew_hbm, out_hbm, dst_v, ew_v, deg_v):
    c = lax.axis_index("c")
    s = lax.axis_index("s")
    wid = c * NS + s
    pltpu.sync_copy(dst_hbm.at[wid], dst_v)
    pltpu.sync_copy(ew_hbm.at[wid], ew_v)

    def zero_body(i, carry):
        deg_v[pl.ds(i * 16, 16)] = jnp.zeros((16,), jnp.float32)
        return carry

    lax.fori_loop(0, N // 16, zero_body, 0, unroll=4)

    def body(g, carry):
        idx = dst_v[pl.ds(g * 16, 16)]
        w = ew_v[pl.ds(g * 16, 16)]
        plsc.addupdate_scatter(deg_v, [idx], w)
        return carry

    lax.fori_loop(0, EPT // 16, body, 0, unroll=8)
    pltpu.sync_copy(deg_v, out_hbm.at[wid])


@functools.partial(
    pl.kernel,
    out_type=jax.ShapeDtypeStruct((NC, N, D_HID), jnp.float32),
    mesh=_mesh,
    compiler_params=_sc_params,
    scratch_types=[
        pltpu.VMEM((2, NSUBC, SUBR), jnp.int32),
        pltpu.VMEM((2, NSUBC, SUBR), jnp.int32),
        pltpu.VMEM((2, NSUBC, SUBR), jnp.float32),
        pltpu.VMEM((2, CHE, D_HID), jnp.float32),
        pltpu.VMEM((RPT, D_HID), jnp.float32),
        pltpu.VMEM_SHARED((N, D_HID), jnp.float32),
        pltpu.VMEM_SHARED((N, D_HID), jnp.float32),
        pltpu.SemaphoreType.DMA,
        pltpu.SemaphoreType.DMA,
        pltpu.SemaphoreType.DMA,
    ],
)
def _agg_kernel(hs_hbm, src_hbm, dst_hbm, ew_hbm, out_hbm,
                srcb, dstb, ewb, rows, zbuf, acc_s, hs_s, seme, semg, sems):
    c = lax.axis_index("c")
    s = lax.axis_index("s")
    wid = c * NS + s

    stage = pltpu.async_copy(hs_hbm.at[pl.ds(s * RPT, RPT)],
                             hs_s.at[pl.ds(s * RPT, RPT)], semg)

    def zero_body(i, carry):
        zbuf[i, :] = jnp.zeros((D_HID,), jnp.float32)
        return carry

    lax.fori_loop(0, RPT, zero_body, 0, unroll=8)
    pltpu.sync_copy(zbuf, acc_s.at[pl.ds(s * RPT, RPT)])
    stage.wait()
    plsc.subcore_barrier()

    row0 = wid * (NCHE * NSUBC)

    def fire_edges(ch, buf):
        rb = row0 + ch * NSUBC
        return [
            pltpu.async_copy(src_hbm.at[pl.ds(rb, NSUBC)], srcb.at[buf], seme),
            pltpu.async_copy(dst_hbm.at[pl.ds(rb, NSUBC)], dstb.at[buf], seme),
            pltpu.async_copy(ew_hbm.at[pl.ds(rb, NSUBC)], ewb.at[buf], seme),
        ]

    edge_pend = {0: fire_edges(0, 0), 1: None}
    scat_pend = {0: [], 1: []}
    for ch in range(NCHE):
        buf = ch % 2
        for d in edge_pend[buf]:
            d.wait()
        for d in scat_pend[buf]:
            d.wait()
        scat_pend[buf] = []
        gathers = [
            pltpu.async_copy(hs_s.at[srcb.at[buf, j]],
                             rows.at[buf, pl.ds(j * SUBR, SUBR)], semg)
            for j in range(NSUBC)
        ]
        if ch + 1 < NCHE:
            for d in scat_pend[1 - buf]:
                d.wait()
            scat_pend[1 - buf] = []
            edge_pend[1 - buf] = fire_edges(ch + 1, 1 - buf)
        for g in gathers:
            g.wait()

        def scale_body(j, carry):
            for q in range(SUBR // 16):
                ew16 = ewb[buf, j, pl.ds(q * 16, 16)]
                for e in range(16):
                    r = j * SUBR + q * 16 + e
                    w = jnp.broadcast_to(ew16[e], (D_HID,))
                    rows[buf, r, :] = rows[buf, r, :] * w
            return carry

        lax.fori_loop(0, NSUBC, scale_body, 0)

        plsc.subcore_barrier()
        scat_pend[buf] = [
            pltpu.async_copy(rows.at[buf, pl.ds(j * SUBR, SUBR)],
                             acc_s.at[dstb.at[buf, j]], sems, add=True)
            for j in range(NSUBC)
        ]

    for d in scat_pend[0] + scat_pend[1]:
        d.wait()
    plsc.subcore_barrier()
    pltpu.sync_copy(acc_s.at[pl.ds(s * RPT, RPT)], zbuf)
    pltpu.sync_copy(zbuf, out_hbm.at[c, pl.ds(s * RPT, RPT)])


def _prep_body(degp_ref, x_ref, w_ref, hs_ref, dinv_ref):
    deg = 1.0 + jnp.sum(degp_ref[...], axis=0)
    di = lax.rsqrt(deg)[:, None]
    dinv_ref[...] = di
    h = jnp.dot(x_ref[...], w_ref[...], preferred_element_type=jnp.float32)
    hs_ref[...] = h * di


@jax.jit
def _prep(degp, x, W1):
    return pl.pallas_call(
        _prep_body,
        out_shape=(
            jax.ShapeDtypeStruct((N, D_HID), jnp.float32),
            jax.ShapeDtypeStruct((N, 1), jnp.float32),
        ),
    )(degp, x, W1)


def _post1_body(p_ref, hs_ref, dinv_ref, b_ref, out_ref):
    di = dinv_ref[...]
    t = p_ref[0] + p_ref[1] + hs_ref[...]
    out1 = jax.nn.relu(di * t + b_ref[...])
    out_ref[...] = di * out1


@jax.jit
def _post1(parts, hs1, dinv, b1r):
    return pl.pallas_call(
        _post1_body,
        out_shape=jax.ShapeDtypeStruct((N, D_HID), jnp.float32),
    )(parts, hs1, dinv, b1r)


def _final_body(p_ref, hs_ref, dinv_ref, w_ref, b_ref, out_ref):
    agg = dinv_ref[...] * (p_ref[0] + p_ref[1] + hs_ref[...])
    logits = jnp.dot(agg, w_ref[...], preferred_element_type=jnp.float32)
    logits = logits + b_ref[...]
    z = logits - jnp.max(logits, axis=1, keepdims=True)
    out_ref[...] = z - jnp.log(jnp.sum(jnp.exp(z), axis=1, keepdims=True))


@jax.jit
def _final(parts, hs2, dinv, W2, b2r):
    return pl.pallas_call(
        _final_body,
        out_shape=jax.ShapeDtypeStruct((N, N_CLS), jnp.float32),
    )(parts, hs2, dinv, W2, b2r)


def kernel(x, edge_index, edge_weight, W1, b1, W2, b2):
    src = edge_index[0]
    dst = edge_index[1]
    degp = _deg_kernel(dst.reshape(NW, EPT), edge_weight.reshape(NW, EPT))
    hs1, dinv = _prep(degp, x, W1)
    pad = E_PAD - E
    zi = jnp.zeros((pad,), jnp.int32)
    srcg = jnp.concatenate([src, zi]).reshape(NROW, SUBR)
    dstg = jnp.concatenate([dst, zi]).reshape(NROW, SUBR)
    ewg = jnp.concatenate(
        [edge_weight, jnp.zeros((pad,), jnp.float32)]).reshape(NROW, SUBR)
    parts1 = _agg_kernel(hs1, srcg, dstg, ewg)
    hs2 = _post1(parts1, hs1, dinv, b1.reshape(1, D_HID))
    parts2 = _agg_kernel(hs2, srcg, dstg, ewg)
    return _final(parts2, hs2, dinv, W2, b2.reshape(1, N_CLS))
